# Initial kernel scaffold; baseline (speedup 1.0000x reference)
import jax
import jax.numpy as jnp
from jax.experimental import pallas as pl


def kernel(token_ids, embedding_weights):
    raise NotImplementedError("write your pallas kernel here")


# speedup vs baseline: 1.4840x; 1.4840x over previous
import functools

import jax
import jax.numpy as jnp
from jax import lax
from jax.experimental import pallas as pl
from jax.experimental.pallas import tpu as pltpu
from jax.experimental.pallas import tpu_sc as plsc

_CHUNK = 128
_STREAMS = 10
_GROUP = _CHUNK * _STREAMS


def _gather_sc(table, idx):
    V, D = table.shape
    N = idx.shape[0]
    info = plsc.get_sparse_core_info()
    nw = info.num_cores * info.num_subcores
    n_per_w = N // nw
    n_chunks = n_per_w // _CHUNK
    n_groups = n_per_w // _GROUP
    assert n_per_w * nw == N and n_groups * _GROUP == n_per_w

    idx3 = idx.reshape(nw, n_chunks, _CHUNK)
    mesh = plsc.VectorSubcoreMesh(core_axis_name="c", subcore_axis_name="s")

    @functools.partial(
        pl.kernel,
        mesh=mesh,
        out_type=jax.ShapeDtypeStruct((N, D), jnp.float32),
        compiler_params=pltpu.CompilerParams(use_tc_tiling_on_sc=False),
        scratch_types=[
            pltpu.VMEM((n_chunks, _CHUNK), jnp.int32),
            pltpu.VMEM((_GROUP, D), jnp.float32),
            pltpu.SemaphoreType.DMA,
        ],
    )
    def k(table_hbm, idx_hbm, out_hbm, idx_v, rows_v, gsem):
        wid = lax.axis_index("s") * info.num_cores + lax.axis_index("c")
        base = wid * n_per_w
        pltpu.sync_copy(idx_hbm.at[wid], idx_v)

        def group(g, carry):
            for t in range(_STREAMS):
                pltpu.async_copy(
                    table_hbm.at[idx_v.at[g * _STREAMS + t]],
                    rows_v.at[pl.ds(t * _CHUNK, _CHUNK)],
                    gsem,
                )
            pltpu.make_async_copy(
                out_hbm.at[pl.ds(0, _GROUP)], rows_v, gsem
            ).wait()
            pltpu.sync_copy(rows_v, out_hbm.at[pl.ds(base + g * _GROUP, _GROUP)])
            return carry

        lax.fori_loop(0, n_groups, group, 0)

    return k(table, idx3)


def kernel(token_ids, embedding_weights):
    B, L = token_ids.shape
    V, D = embedding_weights.shape
    flat = token_ids.reshape(B * L)
    out = _gather_sc(embedding_weights, flat)
    return out.reshape(B, L, D)


# speedup vs baseline: 1.5037x; 1.0133x over previous
import functools

import jax
import jax.numpy as jnp
from jax import lax
from jax.experimental import pallas as pl
from jax.experimental.pallas import tpu as pltpu
from jax.experimental.pallas import tpu_sc as plsc

_CHUNK = 128
_STREAMS = 10
_GROUP = _CHUNK * _STREAMS


def _gather_sc(table, idx):
    V, D = table.shape
    N = idx.shape[0]
    info = plsc.get_sparse_core_info()
    nw = info.num_cores * info.num_subcores
    n_per_w = N // nw
    n_chunks = n_per_w // _CHUNK
    n_groups = n_per_w // _GROUP
    assert n_per_w * nw == N and n_groups * _GROUP == n_per_w
    assert n_groups % 2 == 0

    idx3 = idx.reshape(nw, n_chunks, _CHUNK)
    mesh = plsc.VectorSubcoreMesh(core_axis_name="c", subcore_axis_name="s")

    @functools.partial(
        pl.kernel,
        mesh=mesh,
        out_type=jax.ShapeDtypeStruct((N, D), jnp.float32),
        compiler_params=pltpu.CompilerParams(use_tc_tiling_on_sc=False),
        scratch_types=[
            pltpu.VMEM((n_chunks, _CHUNK), jnp.int32),
            pltpu.VMEM((_GROUP, D), jnp.float32),
            pltpu.VMEM((_GROUP, D), jnp.float32),
            pltpu.SemaphoreType.DMA,
            pltpu.SemaphoreType.DMA,
            pltpu.SemaphoreType.DMA,
            pltpu.SemaphoreType.DMA,
        ],
    )
    def k(table_hbm, idx_hbm, out_hbm, idx_v, rows0, rows1,
          gsem0, gsem1, osem0, osem1):
        wid = lax.axis_index("s") * info.num_cores + lax.axis_index("c")
        base = wid * n_per_w
        pltpu.sync_copy(idx_hbm.at[wid], idx_v)

        def fire_gather(g, rows, gsem):
            for t in range(_STREAMS):
                pltpu.async_copy(
                    table_hbm.at[idx_v.at[g * _STREAMS + t]],
                    rows.at[pl.ds(t * _CHUNK, _CHUNK)],
                    gsem,
                )

        def wait_group_bytes(rows, sem):
            pltpu.make_async_copy(
                out_hbm.at[pl.ds(0, _GROUP)], rows, sem
            ).wait()

        def fire_out(g, rows, osem):
            pltpu.async_copy(
                rows, out_hbm.at[pl.ds(base + g * _GROUP, _GROUP)], osem
            )

        def slot(g, rows_a, gsem_a, osem_a, rows_b, gsem_b, osem_b):
            @pl.when(g + 1 < n_groups)
            def _():
                @pl.when(g >= 1)
                def _():
                    wait_group_bytes(rows_b, osem_b)

                fire_gather(g + 1, rows_b, gsem_b)

            wait_group_bytes(rows_a, gsem_a)
            fire_out(g, rows_a, osem_a)

        fire_gather(0, rows0, gsem0)

        def pair(i, carry):
            slot(2 * i, rows0, gsem0, osem0, rows1, gsem1, osem1)
            slot(2 * i + 1, rows1, gsem1, osem1, rows0, gsem0, osem0)
            return carry

        lax.fori_loop(0, n_groups // 2, pair, 0)
        wait_group_bytes(rows0, osem0)
        wait_group_bytes(rows1, osem1)

    return k(table, idx3)


def kernel(token_ids, embedding_weights):
    B, L = token_ids.shape
    V, D = embedding_weights.shape
    flat = token_ids.reshape(B * L)
    out = _gather_sc(embedding_weights, flat)
    return out.reshape(B, L, D)
